# baseline (speedup 1.0000x reference)
import functools

import jax
import jax.numpy as jnp
from jax import lax
from jax.experimental import pallas as pl
from jax.experimental.pallas import tpu as pltpu
from jax.experimental.pallas import tpu_sc as plsc

_K = 8192
_D = 32
_B = 64
_T = 1024
_N = _B * _T

_info = plsc.get_sparse_core_info()
_NC, _NS = _info.num_cores, _info.num_subcores
_NW = _NC * _NS
_N_PER_W = _N // _NW


@functools.partial(
    pl.kernel,
    mesh=plsc.VectorSubcoreMesh(core_axis_name="c", subcore_axis_name="s"),
    out_type=jax.ShapeDtypeStruct((_B, _T, _D), jnp.float32),
    scratch_types=[
        pltpu.VMEM((_N_PER_W,), jnp.int32),
        pltpu.VMEM((_N_PER_W, _D), jnp.float32),
        pltpu.SemaphoreType.DMA,
    ],
    compiler_params=pltpu.CompilerParams(use_tc_tiling_on_sc=False),
)
def _gather_rows(idx_hbm, table_hbm, out_hbm, idx_v, rows_v, sem):
    wid = lax.axis_index("s") * _NC + lax.axis_index("c")
    row0 = wid * 2
    for r in range(2):
        pltpu.sync_copy(idx_hbm.at[row0 + r], idx_v.at[pl.ds(r * _T, _T)])
    pltpu.async_copy(table_hbm.at[idx_v], rows_v, sem).wait()
    for r in range(2):
        pltpu.sync_copy(rows_v.at[pl.ds(r * _T, _T)], out_hbm.at[row0 + r])


@jax.jit
def kernel(embed_id, weight):
    out = _gather_rows(embed_id, weight)
    one = jnp.float32(1.0) + weight[0, 0] * jnp.float32(0.0)
    return out * one


# speedup vs baseline: 1.3623x; 1.3623x over previous
import functools

import jax
import jax.numpy as jnp
from jax import lax
from jax.experimental import pallas as pl
from jax.experimental.pallas import tpu as pltpu
from jax.experimental.pallas import tpu_sc as plsc

_K = 8192
_D = 32
_B = 64
_T = 1024
_N = _B * _T

_info = plsc.get_sparse_core_info()
_NC, _NS = _info.num_cores, _info.num_subcores
_NW = _NC * _NS
_N_PER_W = _N // _NW


@functools.partial(
    pl.kernel,
    mesh=plsc.VectorSubcoreMesh(core_axis_name="c", subcore_axis_name="s"),
    out_type=jax.ShapeDtypeStruct((_N, _D), jnp.float32),
    scratch_types=[
        pltpu.VMEM((_N_PER_W,), jnp.int32),
        pltpu.VMEM((_N_PER_W, _D), jnp.float32),
        pltpu.SemaphoreType.DMA,
    ],
    compiler_params=pltpu.CompilerParams(use_tc_tiling_on_sc=False),
)
def _gather_rows(idx_hbm, table_hbm, out_hbm, idx_v, rows_v, sem):
    wid = lax.axis_index("s") * _NC + lax.axis_index("c")
    base = wid * _N_PER_W
    pltpu.sync_copy(idx_hbm.at[pl.ds(base, _N_PER_W)], idx_v)
    pltpu.async_copy(table_hbm.at[idx_v], rows_v, sem).wait()
    pltpu.sync_copy(rows_v, out_hbm.at[pl.ds(base, _N_PER_W)])


@jax.jit
def kernel(embed_id, weight):
    flat_ids = embed_id.reshape(_N)
    out = _gather_rows(flat_ids, weight)
    return out.reshape(_B, _T, _D)
